# baseline (speedup 1.0000x reference)
import jax
import jax.numpy as jnp
from jax import lax
from jax.experimental import pallas as pl
from jax.experimental.pallas import tpu as pltpu
from jax.experimental.pallas import tpu_sc as plsc

BATCH = 16384
EMB = 32
NUM_CORES = 2
NUM_SUBCORES = 16
NW = NUM_CORES * NUM_SUBCORES
BPW = BATCH // NW
NSLOT = 8
TILE = 128


def _sc_kernel(user_hbm, item_hbm, uT_hbm, vT_hbm, out_hbm,
               uidx_v, vidx_v, ublk, vblk, outv, sems):
    wid = lax.axis_index("s") * NUM_CORES + lax.axis_index("c")
    base = wid * BPW

    pltpu.sync_copy(user_hbm.at[pl.ds(base, BPW)], uidx_v)
    pltpu.sync_copy(item_hbm.at[pl.ds(base, BPW)], vidx_v)

    lanes = lax.iota(jnp.int32, 16)
    zeros16 = jnp.zeros((16,), jnp.int32)

    def sel(ref, e):
        g = pl.multiple_of((e // 16) * 16, 16)
        vec = ref[pl.ds(g, 16)]
        k = lax.rem(e, 16)
        return lax.reduce_sum(jnp.where(lanes == k, vec, zeros16), axes=(0,))

    def fire(e, slot):
        ru = sel(uidx_v, e)
        rv = sel(vidx_v, e)
        tu = pl.multiple_of((ru // TILE) * TILE, TILE)
        tv = pl.multiple_of((rv // TILE) * TILE, TILE)
        pltpu.async_copy(uT_hbm.at[:, pl.ds(tu, TILE)], ublk.at[slot],
                         sems.at[slot, 0])
        pltpu.async_copy(vT_hbm.at[:, pl.ds(tv, TILE)], vblk.at[slot],
                         sems.at[slot, 1])

    for e in range(NSLOT):
        fire(e, e)

    d_lo = lax.iota(jnp.int32, 16)
    d_hi = d_lo + 16
    lane0 = d_lo == 0

    def body(e, carry):
        slot = lax.rem(e, NSLOT)
        pltpu.make_async_copy(uT_hbm.at[:, pl.ds(0, TILE)], ublk.at[slot],
                              sems.at[slot, 0]).wait()
        pltpu.make_async_copy(vT_hbm.at[:, pl.ds(0, TILE)], vblk.at[slot],
                              sems.at[slot, 1]).wait()
        cu = jnp.full((16,), lax.rem(sel(uidx_v, e), TILE), jnp.int32)
        cv = jnp.full((16,), lax.rem(sel(vidx_v, e), TILE), jnp.int32)
        sv = jnp.full((16,), slot, jnp.int32)
        u_lo = plsc.load_gather(ublk, [sv, d_lo, cu])
        u_hi = plsc.load_gather(ublk, [sv, d_hi, cu])
        v_lo = plsc.load_gather(vblk, [sv, d_lo, cv])
        v_hi = plsc.load_gather(vblk, [sv, d_hi, cv])
        prod = u_lo * v_lo + u_hi * v_hi
        s = lax.reduce_sum(prod, axes=(0,))
        plsc.store_scatter(outv, [jnp.full((16,), e, jnp.int32)],
                           jnp.full((16,), s, jnp.float32), mask=lane0)
        nxt = e + NSLOT

        @pl.when(nxt < BPW)
        def _():
            fire(nxt, slot)

        return carry

    lax.fori_loop(0, BPW, body, 0)

    pltpu.sync_copy(outv, out_hbm.at[pl.ds(base, BPW)])


@jax.jit
def kernel(user, item, user_emb, item_emb):
    user = user.astype(jnp.int32)
    item = item.astype(jnp.int32)
    mesh = plsc.VectorSubcoreMesh(core_axis_name="c", subcore_axis_name="s")
    run = pl.kernel(
        _sc_kernel,
        out_type=jax.ShapeDtypeStruct((BATCH,), jnp.float32),
        mesh=mesh,
        compiler_params=pltpu.CompilerParams(needs_layout_passes=False),
        scratch_types=[
            pltpu.VMEM((BPW,), jnp.int32),
            pltpu.VMEM((BPW,), jnp.int32),
            pltpu.VMEM((NSLOT, EMB, TILE), jnp.float32),
            pltpu.VMEM((NSLOT, EMB, TILE), jnp.float32),
            pltpu.VMEM((BPW,), jnp.float32),
            pltpu.SemaphoreType.DMA((NSLOT, 2)),
        ],
    )
    return run(user, item, user_emb.T, item_emb.T)
